# baseline (speedup 1.0000x reference)
import functools

import jax
import jax.numpy as jnp
from jax import lax
from jax.experimental import pallas as pl
from jax.experimental.pallas import tpu as pltpu
from jax.experimental.pallas import tpu_sc as plsc

NW, NT, ND = 10000, 2000, 4000
D = 128
W = 32
NPASS = D // W

TB_WW = 0
TB_WT, TB_TT, TB_WD, TB_TD = 0, 10000, 12000, 22000
TROWS = 34000
STAGE = 24000
AB_WT, AB_TT, AB_WD, AB_TD = 0, 2000, 4000, 8000
AROWS = 12288
CROWS = AROWS // 128
DUMMY = AROWS - 1

CHUNK = 128
GRP = 16
NCH = 160
NGRP = NCH // GRP
PER_TILE = NCH * CHUNK
SC0_PER_TILE = 320000 // 16
SC1_PER_TILE = 300000 // 16

DRAIN = 125


def _mm_body(x_ref, w_ref, b_ref, o_ref):
    o_ref[...] = (
        jnp.dot(x_ref[...], w_ref[...], preferred_element_type=jnp.float32)
        + b_ref[0:1, :]
    )


def _matmul(x, wt, b8, block_rows):
    n, d = x.shape
    k = wt.shape[1]
    return pl.pallas_call(
        _mm_body,
        grid=(n // block_rows,),
        in_specs=[
            pl.BlockSpec((block_rows, d), lambda i: (i, 0)),
            pl.BlockSpec((d, k), lambda i: (0, 0)),
            pl.BlockSpec((8, k), lambda i: (0, 0)),
        ],
        out_specs=pl.BlockSpec((block_rows, k), lambda i: (i, 0)),
        out_shape=jax.ShapeDtypeStruct((n, k), jnp.float32),
    )(x, wt, b8)


def _bcast2(ref, r, c):
    return plsc.load_gather(
        ref, [jnp.full((16,), r, jnp.int32), jnp.full((16,), c, jnp.int32)]
    )


def _sc_body(tabq, src, dst, wgt, zq, z128,
             h_word, h_topic, h_doc,
             s_0, s_1, d_0, d_1, w_0, w_1, rows, drb, ds_e, ds_o,
             hist, rowidx,
             tab_sh, acc, cnt_sh, gsem_0, gsem_1, ssem_e, ssem_o):
    core = lax.axis_index("c")
    sub = lax.axis_index("s")
    wid = core * 16 + sub
    tbase = wid * NCH
    s_blk = (s_0, s_1)
    d_blk = (d_0, d_1)
    w_blk = (w_0, w_1)
    gsems = (gsem_0, gsem_1)

    def zero_acc():
        pltpu.sync_copy(zq, rows)
        pltpu.sync_copy(zq, drb)
        z0 = sub * (AROWS // 16)
        for k in range(AROWS // 16 // CHUNK):
            pltpu.sync_copy(rows, acc.at[pl.ds(z0 + CHUNK * k, CHUNK), :])

    dummy16 = jnp.full((16,), DUMMY, jnp.int32)
    for k in range(CHUNK // 16):
        ds_e[pl.ds(k * 16, 16)] = dummy16
        ds_o[pl.ds(k * 16, 16)] = dummy16

    pltpu.sync_copy(z128, hist)

    @pl.when(sub == 0)
    def _():
        pltpu.sync_copy(z128, cnt_sh)

    for k in range(CROWS // 16):
        rowidx[pl.ds(k * 16, 16)] = lax.iota(jnp.int32, 16) + (k * 16)

    ones16 = jnp.ones((16,), jnp.float32)
    lane0 = lax.iota(jnp.int32, 16) == 0

    def gfetch(g, b):
        r0 = tbase + GRP * g
        pltpu.async_copy(src.at[pl.ds(r0, GRP), :], s_blk[b], gsems[b])
        pltpu.async_copy(dst.at[pl.ds(r0, GRP), :], d_blk[b], gsems[b])
        pltpu.async_copy(wgt.at[pl.ds(r0, GRP), :], w_blk[b], gsems[b])

    def gwait(g, b):
        r0 = tbase + GRP * g
        pltpu.make_async_copy(
            src.at[pl.ds(r0, GRP), :], s_blk[b], gsems[b]).wait()
        pltpu.make_async_copy(
            dst.at[pl.ds(r0, GRP), :], d_blk[b], gsems[b]).wait()
        pltpu.make_async_copy(
            wgt.at[pl.ds(r0, GRP), :], w_blk[b], gsems[b]).wait()

    def edge_pass(with_hist):
        pltpu.async_copy(rows, acc.at[ds_e], ssem_e, add=True)
        pltpu.async_copy(drb, acc.at[ds_o], ssem_o, add=True)
        gfetch(0, 0)
        gfetch(1, 1)

        def group_pair(i, carry):
            for b in range(2):
                g = 2 * i + b
                gwait(g, b)

                def chunk_pair(j, carry1):
                    for par, buf, dsb, ssem in (
                            (0, rows, ds_e, ssem_e),
                            (1, drb, ds_o, ssem_o)):
                        k = 2 * j + par
                        pltpu.make_async_copy(
                            buf, acc.at[dsb], ssem).wait()
                        pltpu.sync_copy(tab_sh.at[s_blk[b].at[k]], buf)

                        def edge_body(e, carry2):
                            bc = _bcast2(w_blk[b], k, e)
                            if with_hist:
                                d16 = _bcast2(d_blk[b], k, e)
                                plsc.addupdate_scatter(
                                    hist, [d16 >> 7, d16 & 127], ones16,
                                    mask=lane0)
                            for f in range(W // 16):
                                sl = pl.ds(f * 16, 16)
                                buf[e, sl] = buf[e, sl] * bc
                            return carry2

                        lax.fori_loop(0, CHUNK, edge_body, 0, unroll=8)
                        for m in range(CHUNK // 16):
                            sl = pl.ds(m * 16, 16)
                            dsb[sl] = d_blk[b][k, sl]
                        pltpu.async_copy(buf, acc.at[dsb], ssem, add=True)
                    return carry1

                lax.fori_loop(0, GRP // 2, chunk_pair, 0)

                @pl.when(g + 2 < NGRP)
                def _():
                    gfetch(g + 2, b)
            return carry

        lax.fori_loop(0, NGRP // 2, group_pair, 0)
        pltpu.make_async_copy(rows, acc.at[ds_e], ssem_e).wait()
        pltpu.make_async_copy(drb, acc.at[ds_o], ssem_o).wait()

    def mean_rows(dst_ref, q, out_r0, a_r0, b_r0):
        pltpu.sync_copy(acc.at[pl.ds(a_r0, DRAIN), :],
                        rows.at[pl.ds(0, DRAIN), :])
        if b_r0 is not None:
            pltpu.sync_copy(acc.at[pl.ds(b_r0, DRAIN), :],
                            drb.at[pl.ds(0, DRAIN), :])

        def row_body(r, carry):
            aa = a_r0 + r
            ca = jnp.maximum(_bcast2(hist, aa >> 7, aa & 127), 1.0)
            if b_r0 is not None:
                bb = b_r0 + r
                cb = jnp.maximum(_bcast2(hist, bb >> 7, bb & 127), 1.0)
            for f in range(W // 16):
                sl = pl.ds(f * 16, 16)
                v = rows[r, sl] / ca
                if b_r0 is not None:
                    v = v + drb[r, sl] / cb
                rows[r, sl] = v
            return carry

        lax.fori_loop(0, DRAIN, row_body, 0, unroll=5)
        pltpu.sync_copy(rows.at[pl.ds(0, DRAIN), :],
                        dst_ref.at[q].at[pl.ds(out_r0, DRAIN), :])

    def drain(q):
        @pl.when(core == 0)
        def _():
            for k in range(5):
                c = sub * 5 + k
                mean_rows(h_word, q, c * DRAIN, c * DRAIN, None)

        @pl.when(core == 1)
        def _():
            mean_rows(h_topic, q, sub * DRAIN, AB_WT + sub * DRAIN,
                      AB_TT + sub * DRAIN)
            for k in range(2):
                c = sub * 2 + k
                mean_rows(h_doc, q, c * DRAIN, AB_WD + c * DRAIN,
                          AB_TD + c * DRAIN)

    def stage_and_zero(q):
        pltpu.sync_copy(
            tabq.at[q].at[pl.ds(core * (TROWS - STAGE) + sub * (STAGE // 16),
                                STAGE // 16), :],
            tab_sh.at[pl.ds(sub * (STAGE // 16), STAGE // 16), :])
        zero_acc()
        plsc.subcore_barrier()

    stage_and_zero(0)
    edge_pass(with_hist=True)
    pltpu.sync_copy(hist, cnt_sh.at[rowidx], add=True)
    plsc.subcore_barrier()
    pltpu.sync_copy(cnt_sh, hist)
    drain(0)

    def later_pass(q, carry):
        plsc.subcore_barrier()
        stage_and_zero(q)
        edge_pass(with_hist=False)
        plsc.subcore_barrier()
        drain(q)
        return carry

    lax.fori_loop(1, NPASS, later_pass, 0)


_sc_call = functools.partial(
    pl.kernel,
    mesh=plsc.VectorSubcoreMesh(core_axis_name="c", subcore_axis_name="s"),
    compiler_params=pltpu.CompilerParams(
        needs_layout_passes=False, use_tc_tiling_on_sc=False),
    out_type=[
        jax.ShapeDtypeStruct((NPASS, NW, W), jnp.float32),
        jax.ShapeDtypeStruct((NPASS, NT, W), jnp.float32),
        jax.ShapeDtypeStruct((NPASS, ND, W), jnp.float32),
    ],
    scratch_types=[
        pltpu.VMEM((GRP, CHUNK), jnp.int32),
        pltpu.VMEM((GRP, CHUNK), jnp.int32),
        pltpu.VMEM((GRP, CHUNK), jnp.int32),
        pltpu.VMEM((GRP, CHUNK), jnp.int32),
        pltpu.VMEM((GRP, CHUNK), jnp.float32),
        pltpu.VMEM((GRP, CHUNK), jnp.float32),
        pltpu.VMEM((CHUNK, W), jnp.float32),
        pltpu.VMEM((CHUNK, W), jnp.float32),
        pltpu.VMEM((CHUNK,), jnp.int32),
        pltpu.VMEM((CHUNK,), jnp.int32),
        pltpu.VMEM((CROWS, 128), jnp.float32),
        pltpu.VMEM((CROWS,), jnp.int32),
        pltpu.VMEM_SHARED((STAGE, W), jnp.float32),
        pltpu.VMEM_SHARED((AROWS, W), jnp.float32),
        pltpu.VMEM_SHARED((CROWS, 128), jnp.float32),
        pltpu.SemaphoreType.DMA,
        pltpu.SemaphoreType.DMA,
        pltpu.SemaphoreType.DMA,
        pltpu.SemaphoreType.DMA,
    ],
)(_sc_body)


def _tile_pad(a, n_per_tile, val):
    r = a.reshape(16, n_per_tile)
    return jnp.pad(r, ((0, 0), (0, PER_TILE - n_per_tile)),
                   constant_values=val)


def kernel(x_word, x_topic, ei_ww, w_ww, W_ww, b_ww, ei_wt, w_wt, W_wt, b_wt,
           ei_wd, w_wd, W_wd, b_wd, ei_td, w_td, W_td, b_td,
           ei_tt, w_tt, W_tt, b_tt):
    w_word = jnp.concatenate([W_ww, W_wt, W_wd], axis=0).T
    b_word = jnp.broadcast_to(jnp.concatenate([b_ww, b_wt, b_wd]), (8, 384))
    w_top = jnp.concatenate([W_tt, W_td], axis=0).T
    b_top = jnp.broadcast_to(jnp.concatenate([b_tt, b_td]), (8, 256))
    yw = _matmul(x_word, w_word, b_word, 2000)
    yt = _matmul(x_topic, w_top, b_top, 2000)

    table = jnp.concatenate(
        [yw[:, 0:128], yw[:, 128:256], yt[:, 0:128], yw[:, 256:384],
         yt[:, 128:256]], axis=0)
    tabq = jnp.stack([table[:, q * W:(q + 1) * W] for q in range(NPASS)])

    sc1_src = jnp.concatenate([ei_wt[0] + TB_WT, ei_tt[0] + TB_TT,
                               ei_wd[0] + TB_WD, ei_td[0] + TB_TD])
    sc1_dst = jnp.concatenate([ei_wt[1] + AB_WT, ei_tt[1] + AB_TT,
                               ei_wd[1] + AB_WD, ei_td[1] + AB_TD])
    sc1_w = jnp.concatenate([w_wt, w_tt, w_wd, w_td])

    src = jnp.concatenate(
        [_tile_pad(ei_ww[0] + TB_WW, SC0_PER_TILE, 0),
         _tile_pad(sc1_src, SC1_PER_TILE, 0)]).reshape(-1, CHUNK)
    dst = jnp.concatenate(
        [_tile_pad(ei_ww[1], SC0_PER_TILE, DUMMY),
         _tile_pad(sc1_dst, SC1_PER_TILE, DUMMY)]).reshape(-1, CHUNK)
    wgt = jnp.concatenate(
        [_tile_pad(w_ww, SC0_PER_TILE, 0.0),
         _tile_pad(sc1_w, SC1_PER_TILE, 0.0)]).reshape(-1, CHUNK)

    zq = jnp.zeros((CHUNK, W), jnp.float32)
    z128 = jnp.zeros((CROWS, 128), jnp.float32)

    hw4, ht4, hd4 = _sc_call(tabq, src, dst, wgt, zq, z128)
    h_word = jnp.concatenate([hw4[q] for q in range(NPASS)], axis=1)
    h_topic = jnp.concatenate([ht4[q] for q in range(NPASS)], axis=1)
    h_doc = jnp.concatenate([hd4[q] for q in range(NPASS)], axis=1)
    return h_word, h_topic, h_doc


# speedup vs baseline: 1.0011x; 1.0011x over previous
import functools

import jax
import jax.numpy as jnp
from jax import lax
from jax.experimental import pallas as pl
from jax.experimental.pallas import tpu as pltpu
from jax.experimental.pallas import tpu_sc as plsc

NW, NT, ND = 10000, 2000, 4000
D = 128
W = 32
NPASS = D // W

TB_WW = 0
TB_WT, TB_TT, TB_WD, TB_TD = 0, 10000, 12000, 22000
TROWS = 34000
STAGE = 24000
AB_WT, AB_TT, AB_WD, AB_TD = 0, 2000, 4000, 8000
AROWS = 12288
CROWS = AROWS // 128
DUMMY = AROWS - 1

CHUNK = 128
GRP = 8
NCH = 160
NGRP = NCH // GRP
PER_TILE = NCH * CHUNK
SC0_PER_TILE = 320000 // 16
SC1_PER_TILE = 300000 // 16

DRAIN = 125


def _mm_body(x_ref, w_ref, b_ref, o_ref):
    o_ref[...] = (
        jnp.dot(x_ref[...], w_ref[...], preferred_element_type=jnp.float32)
        + b_ref[0:1, :]
    )


def _matmul(x, wt, b8, block_rows):
    n, d = x.shape
    k = wt.shape[1]
    return pl.pallas_call(
        _mm_body,
        grid=(n // block_rows,),
        in_specs=[
            pl.BlockSpec((block_rows, d), lambda i: (i, 0)),
            pl.BlockSpec((d, k), lambda i: (0, 0)),
            pl.BlockSpec((8, k), lambda i: (0, 0)),
        ],
        out_specs=pl.BlockSpec((block_rows, k), lambda i: (i, 0)),
        out_shape=jax.ShapeDtypeStruct((n, k), jnp.float32),
    )(x, wt, b8)


def _bcast2(ref, r, c):
    return plsc.load_gather(
        ref, [jnp.full((16,), r, jnp.int32), jnp.full((16,), c, jnp.int32)]
    )


def _sc_body(tabq, src, dst, wgt, zq, z128,
             h_word, h_topic, h_doc,
             s_0, s_1, d_0, d_1, w_0, w_1, rows, drb, ds_e, ds_o,
             hist, rowidx,
             tab_sh, acc, cnt_sh, gsem_0, gsem_1, ssem_e, ssem_o):
    core = lax.axis_index("c")
    sub = lax.axis_index("s")
    wid = core * 16 + sub
    tbase = wid * NCH
    s_blk = (s_0, s_1)
    d_blk = (d_0, d_1)
    w_blk = (w_0, w_1)
    gsems = (gsem_0, gsem_1)

    def zero_acc():
        pltpu.sync_copy(zq, rows)
        pltpu.sync_copy(zq, drb)
        z0 = sub * (AROWS // 16)
        for k in range(AROWS // 16 // CHUNK):
            pltpu.sync_copy(rows, acc.at[pl.ds(z0 + CHUNK * k, CHUNK), :])

    dummy16 = jnp.full((16,), DUMMY, jnp.int32)
    for k in range(CHUNK // 16):
        ds_e[pl.ds(k * 16, 16)] = dummy16
        ds_o[pl.ds(k * 16, 16)] = dummy16

    pltpu.sync_copy(z128, hist)

    @pl.when(sub == 0)
    def _():
        pltpu.sync_copy(z128, cnt_sh)

    for k in range(CROWS // 16):
        rowidx[pl.ds(k * 16, 16)] = lax.iota(jnp.int32, 16) + (k * 16)

    ones16 = jnp.ones((16,), jnp.float32)
    lane0 = lax.iota(jnp.int32, 16) == 0

    def gfetch(g, b):
        r0 = tbase + GRP * g
        pltpu.async_copy(src.at[pl.ds(r0, GRP), :], s_blk[b], gsems[b])
        pltpu.async_copy(dst.at[pl.ds(r0, GRP), :], d_blk[b], gsems[b])
        pltpu.async_copy(wgt.at[pl.ds(r0, GRP), :], w_blk[b], gsems[b])

    def gwait(g, b):
        r0 = tbase + GRP * g
        pltpu.make_async_copy(
            src.at[pl.ds(r0, GRP), :], s_blk[b], gsems[b]).wait()
        pltpu.make_async_copy(
            dst.at[pl.ds(r0, GRP), :], d_blk[b], gsems[b]).wait()
        pltpu.make_async_copy(
            wgt.at[pl.ds(r0, GRP), :], w_blk[b], gsems[b]).wait()

    def edge_pass(with_hist):
        pltpu.async_copy(rows, acc.at[ds_e], ssem_e, add=True)
        pltpu.async_copy(drb, acc.at[ds_o], ssem_o, add=True)
        gfetch(0, 0)
        gfetch(1, 1)

        def group_pair(i, carry):
            for b in range(2):
                g = 2 * i + b
                gwait(g, b)

                def chunk_pair(j, carry1):
                    for par, buf, dsb, ssem in (
                            (0, rows, ds_e, ssem_e),
                            (1, drb, ds_o, ssem_o)):
                        k = 2 * j + par
                        pltpu.make_async_copy(
                            buf, acc.at[dsb], ssem).wait()
                        pltpu.sync_copy(tab_sh.at[s_blk[b].at[k]], buf)

                        def edge_body(e, carry2):
                            bc = _bcast2(w_blk[b], k, e)
                            if with_hist:
                                d16 = _bcast2(d_blk[b], k, e)
                                plsc.addupdate_scatter(
                                    hist, [d16 >> 7, d16 & 127], ones16,
                                    mask=lane0)
                            for f in range(W // 16):
                                sl = pl.ds(f * 16, 16)
                                buf[e, sl] = buf[e, sl] * bc
                            return carry2

                        lax.fori_loop(0, CHUNK, edge_body, 0, unroll=8)
                        for m in range(CHUNK // 16):
                            sl = pl.ds(m * 16, 16)
                            dsb[sl] = d_blk[b][k, sl]
                        pltpu.async_copy(buf, acc.at[dsb], ssem, add=True)
                    return carry1

                lax.fori_loop(0, GRP // 2, chunk_pair, 0)

                @pl.when(g + 2 < NGRP)
                def _():
                    gfetch(g + 2, b)
            return carry

        lax.fori_loop(0, NGRP // 2, group_pair, 0)
        pltpu.make_async_copy(rows, acc.at[ds_e], ssem_e).wait()
        pltpu.make_async_copy(drb, acc.at[ds_o], ssem_o).wait()

    def mean_rows(dst_ref, q, out_r0, a_r0, b_r0):
        pltpu.sync_copy(acc.at[pl.ds(a_r0, DRAIN), :],
                        rows.at[pl.ds(0, DRAIN), :])
        if b_r0 is not None:
            pltpu.sync_copy(acc.at[pl.ds(b_r0, DRAIN), :],
                            drb.at[pl.ds(0, DRAIN), :])

        def row_body(r, carry):
            aa = a_r0 + r
            ca = jnp.maximum(_bcast2(hist, aa >> 7, aa & 127), 1.0)
            if b_r0 is not None:
                bb = b_r0 + r
                cb = jnp.maximum(_bcast2(hist, bb >> 7, bb & 127), 1.0)
            for f in range(W // 16):
                sl = pl.ds(f * 16, 16)
                v = rows[r, sl] / ca
                if b_r0 is not None:
                    v = v + drb[r, sl] / cb
                rows[r, sl] = v
            return carry

        lax.fori_loop(0, DRAIN, row_body, 0, unroll=5)
        pltpu.sync_copy(rows.at[pl.ds(0, DRAIN), :],
                        dst_ref.at[q].at[pl.ds(out_r0, DRAIN), :])

    def drain(q):
        @pl.when(core == 0)
        def _():
            for k in range(5):
                c = sub * 5 + k
                mean_rows(h_word, q, c * DRAIN, c * DRAIN, None)

        @pl.when(core == 1)
        def _():
            mean_rows(h_topic, q, sub * DRAIN, AB_WT + sub * DRAIN,
                      AB_TT + sub * DRAIN)
            for k in range(2):
                c = sub * 2 + k
                mean_rows(h_doc, q, c * DRAIN, AB_WD + c * DRAIN,
                          AB_TD + c * DRAIN)

    def stage_and_zero(q):
        pltpu.sync_copy(
            tabq.at[q].at[pl.ds(core * (TROWS - STAGE) + sub * (STAGE // 16),
                                STAGE // 16), :],
            tab_sh.at[pl.ds(sub * (STAGE // 16), STAGE // 16), :])
        zero_acc()
        plsc.subcore_barrier()

    stage_and_zero(0)
    edge_pass(with_hist=True)
    pltpu.sync_copy(hist, cnt_sh.at[rowidx], add=True)
    plsc.subcore_barrier()
    pltpu.sync_copy(cnt_sh, hist)
    drain(0)

    def later_pass(q, carry):
        plsc.subcore_barrier()
        stage_and_zero(q)
        edge_pass(with_hist=False)
        plsc.subcore_barrier()
        drain(q)
        return carry

    lax.fori_loop(1, NPASS, later_pass, 0)


_sc_call = functools.partial(
    pl.kernel,
    mesh=plsc.VectorSubcoreMesh(core_axis_name="c", subcore_axis_name="s"),
    compiler_params=pltpu.CompilerParams(
        needs_layout_passes=False, use_tc_tiling_on_sc=False),
    out_type=[
        jax.ShapeDtypeStruct((NPASS, NW, W), jnp.float32),
        jax.ShapeDtypeStruct((NPASS, NT, W), jnp.float32),
        jax.ShapeDtypeStruct((NPASS, ND, W), jnp.float32),
    ],
    scratch_types=[
        pltpu.VMEM((GRP, CHUNK), jnp.int32),
        pltpu.VMEM((GRP, CHUNK), jnp.int32),
        pltpu.VMEM((GRP, CHUNK), jnp.int32),
        pltpu.VMEM((GRP, CHUNK), jnp.int32),
        pltpu.VMEM((GRP, CHUNK), jnp.float32),
        pltpu.VMEM((GRP, CHUNK), jnp.float32),
        pltpu.VMEM((CHUNK, W), jnp.float32),
        pltpu.VMEM((CHUNK, W), jnp.float32),
        pltpu.VMEM((CHUNK,), jnp.int32),
        pltpu.VMEM((CHUNK,), jnp.int32),
        pltpu.VMEM((CROWS, 128), jnp.float32),
        pltpu.VMEM((CROWS,), jnp.int32),
        pltpu.VMEM_SHARED((STAGE, W), jnp.float32),
        pltpu.VMEM_SHARED((AROWS, W), jnp.float32),
        pltpu.VMEM_SHARED((CROWS, 128), jnp.float32),
        pltpu.SemaphoreType.DMA,
        pltpu.SemaphoreType.DMA,
        pltpu.SemaphoreType.DMA,
        pltpu.SemaphoreType.DMA,
    ],
)(_sc_body)


def _tile_pad(a, n_per_tile, val):
    r = a.reshape(16, n_per_tile)
    return jnp.pad(r, ((0, 0), (0, PER_TILE - n_per_tile)),
                   constant_values=val)


def kernel(x_word, x_topic, ei_ww, w_ww, W_ww, b_ww, ei_wt, w_wt, W_wt, b_wt,
           ei_wd, w_wd, W_wd, b_wd, ei_td, w_td, W_td, b_td,
           ei_tt, w_tt, W_tt, b_tt):
    w_word = jnp.concatenate([W_ww, W_wt, W_wd], axis=0).T
    b_word = jnp.broadcast_to(jnp.concatenate([b_ww, b_wt, b_wd]), (8, 384))
    w_top = jnp.concatenate([W_tt, W_td], axis=0).T
    b_top = jnp.broadcast_to(jnp.concatenate([b_tt, b_td]), (8, 256))
    yw = _matmul(x_word, w_word, b_word, 2000)
    yt = _matmul(x_topic, w_top, b_top, 2000)

    table = jnp.concatenate(
        [yw[:, 0:128], yw[:, 128:256], yt[:, 0:128], yw[:, 256:384],
         yt[:, 128:256]], axis=0)
    tabq = jnp.stack([table[:, q * W:(q + 1) * W] for q in range(NPASS)])

    sc1_src = jnp.concatenate([ei_wt[0] + TB_WT, ei_tt[0] + TB_TT,
                               ei_wd[0] + TB_WD, ei_td[0] + TB_TD])
    sc1_dst = jnp.concatenate([ei_wt[1] + AB_WT, ei_tt[1] + AB_TT,
                               ei_wd[1] + AB_WD, ei_td[1] + AB_TD])
    sc1_w = jnp.concatenate([w_wt, w_tt, w_wd, w_td])

    src = jnp.concatenate(
        [_tile_pad(ei_ww[0] + TB_WW, SC0_PER_TILE, 0),
         _tile_pad(sc1_src, SC1_PER_TILE, 0)]).reshape(-1, CHUNK)
    dst = jnp.concatenate(
        [_tile_pad(ei_ww[1], SC0_PER_TILE, DUMMY),
         _tile_pad(sc1_dst, SC1_PER_TILE, DUMMY)]).reshape(-1, CHUNK)
    wgt = jnp.concatenate(
        [_tile_pad(w_ww, SC0_PER_TILE, 0.0),
         _tile_pad(sc1_w, SC1_PER_TILE, 0.0)]).reshape(-1, CHUNK)

    zq = jnp.zeros((CHUNK, W), jnp.float32)
    z128 = jnp.zeros((CROWS, 128), jnp.float32)

    hw4, ht4, hd4 = _sc_call(tabq, src, dst, wgt, zq, z128)
    h_word = jnp.concatenate([hw4[q] for q in range(NPASS)], axis=1)
    h_topic = jnp.concatenate([ht4[q] for q in range(NPASS)], axis=1)
    h_doc = jnp.concatenate([hd4[q] for q in range(NPASS)], axis=1)
    return h_word, h_topic, h_doc


# speedup vs baseline: 1.0719x; 1.0706x over previous
import functools

import jax
import jax.numpy as jnp
from jax import lax
from jax.experimental import pallas as pl
from jax.experimental.pallas import tpu as pltpu
from jax.experimental.pallas import tpu_sc as plsc

NW, NT, ND = 10000, 2000, 4000
D = 128
W = 32
NPASS = D // W

TB_WW = 0
TB_WT, TB_TT, TB_WD, TB_TD = 0, 10000, 12000, 22000
TROWS = 34000
STAGE = 24000
AB_WT, AB_TT, AB_WD, AB_TD = 0, 2000, 4000, 8000
AROWS = 12288
CROWS = AROWS // 128
DUMMY = AROWS - 1

CHUNK = 128
GRP = 8
NCH = 160
NGRP = NCH // GRP
PER_TILE = NCH * CHUNK
SC0_PER_TILE = 320000 // 16
SC1_PER_TILE = 300000 // 16

DRAIN = 125


def _mm_body(x_ref, w_ref, b_ref, o_ref):
    o_ref[...] = (
        jnp.dot(x_ref[...], w_ref[...], preferred_element_type=jnp.float32)
        + b_ref[0:1, :]
    )


def _matmul(x, wt, b8, block_rows):
    n, d = x.shape
    k = wt.shape[1]
    return pl.pallas_call(
        _mm_body,
        grid=(n // block_rows,),
        in_specs=[
            pl.BlockSpec((block_rows, d), lambda i: (i, 0)),
            pl.BlockSpec((d, k), lambda i: (0, 0)),
            pl.BlockSpec((8, k), lambda i: (0, 0)),
        ],
        out_specs=pl.BlockSpec((block_rows, k), lambda i: (i, 0)),
        out_shape=jax.ShapeDtypeStruct((n, k), jnp.float32),
    )(x, wt, b8)


def _bcast2(ref, r, c):
    return plsc.load_gather(
        ref, [jnp.full((16,), r, jnp.int32), jnp.full((16,), c, jnp.int32)]
    )


def _sc_body(tabq, src, dst, wgt, zq, z128,
             h_word, h_topic, h_doc,
             s_0, s_1, d_0, d_1, w_0, w_1, rows, drb, ds_e, ds_o,
             hist, rowidx,
             tab_sh, acc, cnt_sh, gsem_0, gsem_1, ssem_e, ssem_o):
    core = lax.axis_index("c")
    sub = lax.axis_index("s")
    wid = core * 16 + sub
    tbase = wid * NCH
    s_blk = (s_0, s_1)
    d_blk = (d_0, d_1)
    w_blk = (w_0, w_1)
    gsems = (gsem_0, gsem_1)

    def zero_acc():
        pltpu.sync_copy(zq, rows)
        pltpu.sync_copy(zq, drb)
        z0 = sub * (AROWS // 16)
        for k in range(AROWS // 16 // CHUNK):
            pltpu.sync_copy(rows, acc.at[pl.ds(z0 + CHUNK * k, CHUNK), :])

    dummy16 = jnp.full((16,), DUMMY, jnp.int32)
    for k in range(CHUNK // 16):
        ds_e[pl.ds(k * 16, 16)] = dummy16
        ds_o[pl.ds(k * 16, 16)] = dummy16

    pltpu.sync_copy(z128, hist)

    @pl.when(sub == 0)
    def _():
        pltpu.sync_copy(z128, cnt_sh)

    for k in range(CROWS // 16):
        rowidx[pl.ds(k * 16, 16)] = lax.iota(jnp.int32, 16) + (k * 16)

    ones16 = jnp.ones((16,), jnp.float32)
    lane0 = lax.iota(jnp.int32, 16) == 0

    def gfetch(g, b):
        r0 = tbase + GRP * g
        pltpu.async_copy(src.at[pl.ds(r0, GRP), :], s_blk[b], gsems[b])
        pltpu.async_copy(dst.at[pl.ds(r0, GRP), :], d_blk[b], gsems[b])
        pltpu.async_copy(wgt.at[pl.ds(r0, GRP), :], w_blk[b], gsems[b])

    def gwait(g, b):
        r0 = tbase + GRP * g
        pltpu.make_async_copy(
            src.at[pl.ds(r0, GRP), :], s_blk[b], gsems[b]).wait()
        pltpu.make_async_copy(
            dst.at[pl.ds(r0, GRP), :], d_blk[b], gsems[b]).wait()
        pltpu.make_async_copy(
            wgt.at[pl.ds(r0, GRP), :], w_blk[b], gsems[b]).wait()

    def edge_pass(with_hist):
        pltpu.async_copy(rows, acc.at[ds_e], ssem_e, add=True)
        pltpu.async_copy(drb, acc.at[ds_o], ssem_o, add=True)
        gfetch(0, 0)
        gfetch(1, 1)

        def group_pair(i, carry):
            for b in range(2):
                g = 2 * i + b
                gwait(g, b)

                def chunk_pair(j, carry1):
                    for par, buf, dsb, ssem in (
                            (0, rows, ds_e, ssem_e),
                            (1, drb, ds_o, ssem_o)):
                        k = 2 * j + par
                        pltpu.make_async_copy(
                            buf, acc.at[dsb], ssem).wait()
                        pltpu.sync_copy(tab_sh.at[s_blk[b].at[k]], buf)

                        def edge_body(e, carry2):
                            bc = _bcast2(w_blk[b], k, e)
                            if with_hist:
                                d16 = _bcast2(d_blk[b], k, e)
                                plsc.addupdate_scatter(
                                    hist, [d16 >> 7, d16 & 127], ones16,
                                    mask=lane0)
                            for f in range(W // 16):
                                sl = pl.ds(f * 16, 16)
                                buf[e, sl] = buf[e, sl] * bc
                            return carry2

                        lax.fori_loop(0, CHUNK, edge_body, 0, unroll=4)
                        for m in range(CHUNK // 16):
                            sl = pl.ds(m * 16, 16)
                            dsb[sl] = d_blk[b][k, sl]
                        pltpu.async_copy(buf, acc.at[dsb], ssem, add=True)
                    return carry1

                lax.fori_loop(0, GRP // 2, chunk_pair, 0)

                @pl.when(g + 2 < NGRP)
                def _():
                    gfetch(g + 2, b)
            return carry

        lax.fori_loop(0, NGRP // 2, group_pair, 0)
        pltpu.make_async_copy(rows, acc.at[ds_e], ssem_e).wait()
        pltpu.make_async_copy(drb, acc.at[ds_o], ssem_o).wait()

    def mean_rows(dst_ref, q, out_r0, a_r0, b_r0):
        pltpu.sync_copy(acc.at[pl.ds(a_r0, DRAIN), :],
                        rows.at[pl.ds(0, DRAIN), :])
        if b_r0 is not None:
            pltpu.sync_copy(acc.at[pl.ds(b_r0, DRAIN), :],
                            drb.at[pl.ds(0, DRAIN), :])

        def row_body(r, carry):
            aa = a_r0 + r
            ca = jnp.maximum(_bcast2(hist, aa >> 7, aa & 127), 1.0)
            if b_r0 is not None:
                bb = b_r0 + r
                cb = jnp.maximum(_bcast2(hist, bb >> 7, bb & 127), 1.0)
            for f in range(W // 16):
                sl = pl.ds(f * 16, 16)
                v = rows[r, sl] / ca
                if b_r0 is not None:
                    v = v + drb[r, sl] / cb
                rows[r, sl] = v
            return carry

        lax.fori_loop(0, DRAIN, row_body, 0, unroll=5)
        pltpu.sync_copy(rows.at[pl.ds(0, DRAIN), :],
                        dst_ref.at[q].at[pl.ds(out_r0, DRAIN), :])

    def drain(q):
        @pl.when(core == 0)
        def _():
            for k in range(5):
                c = sub * 5 + k
                mean_rows(h_word, q, c * DRAIN, c * DRAIN, None)

        @pl.when(core == 1)
        def _():
            mean_rows(h_topic, q, sub * DRAIN, AB_WT + sub * DRAIN,
                      AB_TT + sub * DRAIN)
            for k in range(2):
                c = sub * 2 + k
                mean_rows(h_doc, q, c * DRAIN, AB_WD + c * DRAIN,
                          AB_TD + c * DRAIN)

    def stage_and_zero(q):
        pltpu.sync_copy(
            tabq.at[q].at[pl.ds(core * (TROWS - STAGE) + sub * (STAGE // 16),
                                STAGE // 16), :],
            tab_sh.at[pl.ds(sub * (STAGE // 16), STAGE // 16), :])
        zero_acc()
        plsc.subcore_barrier()

    stage_and_zero(0)
    edge_pass(with_hist=True)
    pltpu.sync_copy(hist, cnt_sh.at[rowidx], add=True)
    plsc.subcore_barrier()
    pltpu.sync_copy(cnt_sh, hist)
    drain(0)

    def later_pass(q, carry):
        plsc.subcore_barrier()
        stage_and_zero(q)
        edge_pass(with_hist=False)
        plsc.subcore_barrier()
        drain(q)
        return carry

    lax.fori_loop(1, NPASS, later_pass, 0)


_sc_call = functools.partial(
    pl.kernel,
    mesh=plsc.VectorSubcoreMesh(core_axis_name="c", subcore_axis_name="s"),
    compiler_params=pltpu.CompilerParams(
        needs_layout_passes=False, use_tc_tiling_on_sc=False),
    out_type=[
        jax.ShapeDtypeStruct((NPASS, NW, W), jnp.float32),
        jax.ShapeDtypeStruct((NPASS, NT, W), jnp.float32),
        jax.ShapeDtypeStruct((NPASS, ND, W), jnp.float32),
    ],
    scratch_types=[
        pltpu.VMEM((GRP, CHUNK), jnp.int32),
        pltpu.VMEM((GRP, CHUNK), jnp.int32),
        pltpu.VMEM((GRP, CHUNK), jnp.int32),
        pltpu.VMEM((GRP, CHUNK), jnp.int32),
        pltpu.VMEM((GRP, CHUNK), jnp.float32),
        pltpu.VMEM((GRP, CHUNK), jnp.float32),
        pltpu.VMEM((CHUNK, W), jnp.float32),
        pltpu.VMEM((CHUNK, W), jnp.float32),
        pltpu.VMEM((CHUNK,), jnp.int32),
        pltpu.VMEM((CHUNK,), jnp.int32),
        pltpu.VMEM((CROWS, 128), jnp.float32),
        pltpu.VMEM((CROWS,), jnp.int32),
        pltpu.VMEM_SHARED((STAGE, W), jnp.float32),
        pltpu.VMEM_SHARED((AROWS, W), jnp.float32),
        pltpu.VMEM_SHARED((CROWS, 128), jnp.float32),
        pltpu.SemaphoreType.DMA,
        pltpu.SemaphoreType.DMA,
        pltpu.SemaphoreType.DMA,
        pltpu.SemaphoreType.DMA,
    ],
)(_sc_body)


def _tile_pad(a, n_per_tile, val):
    r = a.reshape(16, n_per_tile)
    return jnp.pad(r, ((0, 0), (0, PER_TILE - n_per_tile)),
                   constant_values=val)


def kernel(x_word, x_topic, ei_ww, w_ww, W_ww, b_ww, ei_wt, w_wt, W_wt, b_wt,
           ei_wd, w_wd, W_wd, b_wd, ei_td, w_td, W_td, b_td,
           ei_tt, w_tt, W_tt, b_tt):
    w_word = jnp.concatenate([W_ww, W_wt, W_wd], axis=0).T
    b_word = jnp.broadcast_to(jnp.concatenate([b_ww, b_wt, b_wd]), (8, 384))
    w_top = jnp.concatenate([W_tt, W_td], axis=0).T
    b_top = jnp.broadcast_to(jnp.concatenate([b_tt, b_td]), (8, 256))
    yw = _matmul(x_word, w_word, b_word, 2000)
    yt = _matmul(x_topic, w_top, b_top, 2000)

    table = jnp.concatenate(
        [yw[:, 0:128], yw[:, 128:256], yt[:, 0:128], yw[:, 256:384],
         yt[:, 128:256]], axis=0)
    tabq = jnp.stack([table[:, q * W:(q + 1) * W] for q in range(NPASS)])

    sc1_src = jnp.concatenate([ei_wt[0] + TB_WT, ei_tt[0] + TB_TT,
                               ei_wd[0] + TB_WD, ei_td[0] + TB_TD])
    sc1_dst = jnp.concatenate([ei_wt[1] + AB_WT, ei_tt[1] + AB_TT,
                               ei_wd[1] + AB_WD, ei_td[1] + AB_TD])
    sc1_w = jnp.concatenate([w_wt, w_tt, w_wd, w_td])

    src = jnp.concatenate(
        [_tile_pad(ei_ww[0] + TB_WW, SC0_PER_TILE, 0),
         _tile_pad(sc1_src, SC1_PER_TILE, 0)]).reshape(-1, CHUNK)
    dst = jnp.concatenate(
        [_tile_pad(ei_ww[1], SC0_PER_TILE, DUMMY),
         _tile_pad(sc1_dst, SC1_PER_TILE, DUMMY)]).reshape(-1, CHUNK)
    wgt = jnp.concatenate(
        [_tile_pad(w_ww, SC0_PER_TILE, 0.0),
         _tile_pad(sc1_w, SC1_PER_TILE, 0.0)]).reshape(-1, CHUNK)

    zq = jnp.zeros((CHUNK, W), jnp.float32)
    z128 = jnp.zeros((CROWS, 128), jnp.float32)

    hw4, ht4, hd4 = _sc_call(tabq, src, dst, wgt, zq, z128)
    h_word = jnp.concatenate([hw4[q] for q in range(NPASS)], axis=1)
    h_topic = jnp.concatenate([ht4[q] for q in range(NPASS)], axis=1)
    h_doc = jnp.concatenate([hd4[q] for q in range(NPASS)], axis=1)
    return h_word, h_topic, h_doc
